# baseline (speedup 1.0000x reference)
import jax
import jax.numpy as jnp
from jax import lax
from jax.experimental import pallas as pl
from jax.experimental.pallas import tpu as pltpu
from jax.experimental.pallas import tpu_sc as plsc

N = 524288
K = 16
NC = 2
NS = 16
L = 16
NW = NC * NS
ROWS_W = N // NW
C = 1024
NCHUNK = ROWS_W // C
CW = C // 128
NWD = N // 128


def _body(x_hbm, t_hbm, y_hbm, d_hbm, out_hbm,
          t_v0, y_v0, d_v0, x_v0, o_v0,
          t_v1, y_v1, d_v1, x_v1, o_v1,
          sem_in0, sem_in1, sem_out0, sem_out1):
    wid = lax.axis_index("s") * NC + lax.axis_index("c")
    lane = lax.broadcasted_iota(jnp.int32, (L,), 0)
    w_base = wid * ROWS_W
    w_bw = wid * (ROWS_W // 128)

    bufs = [(t_v0, y_v0, d_v0, x_v0, o_v0, sem_in0, sem_out0),
            (t_v1, y_v1, d_v1, x_v1, o_v1, sem_in1, sem_out1)]

    def start_in(ci):
        t_v, y_v, d_v, x_v, _, sem_in, _ = bufs[ci & 1]
        base = pl.multiple_of(w_base + ci * C, C)
        bw = pl.multiple_of(w_bw + ci * CW, 8)
        src = lambda h: h.at[:, pl.ds(bw, CW), :, :]
        return [pltpu.async_copy(src(t_hbm), t_v, sem_in),
                pltpu.async_copy(src(y_hbm), y_v, sem_in),
                pltpu.async_copy(src(d_hbm), d_v, sem_in),
                pltpu.async_copy(x_hbm.at[pl.ds(base, C)], x_v, sem_in)]

    def compute(ci):
        t_v, y_v, d_v, x_v, o_v, _, _ = bufs[ci & 1]

        @pl.loop(0, C // L, unroll=4)
        def _grp(g):
            rhigh = jnp.full((L,), g >> 3, jnp.int32)
            rlow = (g * L) % 128 + lane
            xv = x_v[pl.ds(g * L, L)]

            def gat(ref, j):
                return plsc.load_gather(ref, [j >> 3, rhigh, j & 7, rlow])

            c = jnp.zeros((L,), jnp.int32)
            for s in (8, 4, 2, 1):
                c = c + jnp.where(xv > gat(t_v, c + (s - 1)), s, 0)
            c = c + jnp.where(xv > gat(t_v, c), 1, 0)

            eq0 = c == 0
            eq1 = c == K
            k0 = jnp.where(eq0, 0, jnp.where(eq1, K - 2, c - 1))
            k1 = jnp.where(eq0, 1, jnp.where(eq1, K - 1, c))
            interior = jnp.logical_not(jnp.logical_or(eq0, eq1))

            t0 = gat(t_v, k0)
            t1 = gat(t_v, k1)
            y0 = gat(y_v, k0)
            y1 = gat(y_v, k1)
            d0 = gat(d_v, k0)
            d1 = gat(d_v, k1)

            dt = t1 - t0
            dy = y1 - y0
            sl = dy / dt
            e = (xv - t0) / dt
            ome = 1.0 - e
            n0 = dy * (sl * e * e + d0 * e * ome)
            n1 = sl + (d1 + d0 - 2.0 * sl) * e * ome
            n1s = jnp.where(interior, n1, 1.0)
            p = y0 + n0 / n1s
            p = jnp.where(interior, p, xv)
            p = jnp.where(eq0, d0 * xv + (y0 - d0 * t0), p)
            p = jnp.where(eq1, d1 * xv + (y1 - d1 * t1), p)
            o_v[pl.ds(g * L, L)] = p

    in_descs = {}
    out_descs = {}
    in_descs[0] = start_in(0)
    for ci in range(NCHUNK):
        if ci + 1 < NCHUNK:
            in_descs[ci + 1] = start_in(ci + 1)
        for desc in in_descs.pop(ci):
            desc.wait()
        compute(ci)
        if ci >= 2:
            out_descs.pop(ci - 2).wait()
        o_v = bufs[ci & 1][4]
        sem_out = bufs[ci & 1][6]
        base = pl.multiple_of(w_base + ci * C, C)
        out_descs[ci] = pltpu.async_copy(o_v, out_hbm.at[pl.ds(base, C)],
                                         sem_out)
    for ci in sorted(out_descs):
        out_descs.pop(ci).wait()


def _tileview(a):
    return a.T.reshape(K // 8, 8, NWD, 128).transpose(0, 2, 1, 3)


def kernel(x, t, y, d):
    xf = x.reshape(N)
    mesh = plsc.VectorSubcoreMesh(
        core_axis_name="c", subcore_axis_name="s", num_cores=NC, num_subcores=NS
    )
    tbl = pltpu.VMEM((K // 8, CW, 8, 128), jnp.float32)
    vec = pltpu.VMEM((C,), jnp.float32)
    out = pl.kernel(
        _body,
        out_type=jax.ShapeDtypeStruct((N,), jnp.float32),
        mesh=mesh,
        compiler_params=pltpu.CompilerParams(needs_layout_passes=False),
        scratch_types=[
            tbl, tbl, tbl, vec, vec,
            tbl, tbl, tbl, vec, vec,
            pltpu.SemaphoreType.DMA, pltpu.SemaphoreType.DMA,
            pltpu.SemaphoreType.DMA, pltpu.SemaphoreType.DMA,
        ],
    )(xf, _tileview(t), _tileview(y), _tileview(d))
    return out[:, None]


# speedup vs baseline: 1.5169x; 1.5169x over previous
import jax
import jax.numpy as jnp
from jax import lax
from jax.experimental import pallas as pl
from jax.experimental.pallas import tpu as pltpu
from jax.experimental.pallas import tpu_sc as plsc

N = 524288
K = 16
NC = 2
NS = 16
L = 16
NW = NC * NS
ROWS_W = N // NW
C = 1024
NCHUNK = ROWS_W // C
CW = C // 128
NWD = N // 128


def _body(x_hbm, t_hbm, y_hbm, d_hbm, out_hbm,
          t_v0, y_v0, d_v0, x_v0, o_v0,
          t_v1, y_v1, d_v1, x_v1, o_v1,
          sem_in0, sem_in1, sem_out0, sem_out1):
    wid = lax.axis_index("s") * NC + lax.axis_index("c")
    lane = lax.broadcasted_iota(jnp.int32, (L,), 0)
    w_base = wid * ROWS_W
    w_bw = wid * (ROWS_W // 128)

    bufs = [(t_v0, y_v0, d_v0, x_v0, o_v0, sem_in0, sem_out0),
            (t_v1, y_v1, d_v1, x_v1, o_v1, sem_in1, sem_out1)]

    def start_in(ci):
        t_v, y_v, d_v, x_v, _, sem_in, _ = bufs[ci & 1]
        base = pl.multiple_of(w_base + ci * C, C)
        bw = pl.multiple_of(w_bw + ci * CW, 8)
        src = lambda h: h.at[:, pl.ds(bw, CW), :, :]
        return [pltpu.async_copy(src(t_hbm), t_v, sem_in),
                pltpu.async_copy(src(y_hbm), y_v, sem_in),
                pltpu.async_copy(src(d_hbm), d_v, sem_in),
                pltpu.async_copy(x_hbm.at[pl.ds(base, C)], x_v, sem_in)]

    def compute(ci):
        t_v, y_v, d_v, x_v, o_v, _, _ = bufs[ci & 1]

        @plsc.parallel_loop(0, C // L, unroll=4)
        def _grp(g):
            rhigh = jnp.full((L,), g >> 3, jnp.int32)
            rlow = (g * L) % 128 + lane
            xv = x_v[pl.ds(g * L, L)]

            def gat(ref, j):
                return plsc.load_gather(ref, [j >> 3, rhigh, j & 7, rlow])

            c = jnp.zeros((L,), jnp.int32)
            for s in (8, 4, 2, 1):
                c = c + jnp.where(xv > gat(t_v, c + (s - 1)), s, 0)
            c = c + jnp.where(xv > gat(t_v, c), 1, 0)

            eq0 = c == 0
            eq1 = c == K
            k0 = jnp.where(eq0, 0, jnp.where(eq1, K - 2, c - 1))
            k1 = jnp.where(eq0, 1, jnp.where(eq1, K - 1, c))
            interior = jnp.logical_not(jnp.logical_or(eq0, eq1))

            t0 = gat(t_v, k0)
            t1 = gat(t_v, k1)
            y0 = gat(y_v, k0)
            y1 = gat(y_v, k1)
            d0 = gat(d_v, k0)
            d1 = gat(d_v, k1)

            dt = t1 - t0
            dy = y1 - y0
            sl = dy / dt
            e = (xv - t0) / dt
            ome = 1.0 - e
            n0 = dy * (sl * e * e + d0 * e * ome)
            n1 = sl + (d1 + d0 - 2.0 * sl) * e * ome
            n1s = jnp.where(interior, n1, 1.0)
            p = y0 + n0 / n1s
            p = jnp.where(interior, p, xv)
            p = jnp.where(eq0, d0 * xv + (y0 - d0 * t0), p)
            p = jnp.where(eq1, d1 * xv + (y1 - d1 * t1), p)
            o_v[pl.ds(g * L, L)] = p

    in_descs = {}
    out_descs = {}
    in_descs[0] = start_in(0)
    for ci in range(NCHUNK):
        if ci + 1 < NCHUNK:
            in_descs[ci + 1] = start_in(ci + 1)
        for desc in in_descs.pop(ci):
            desc.wait()
        compute(ci)
        if ci >= 2:
            out_descs.pop(ci - 2).wait()
        o_v = bufs[ci & 1][4]
        sem_out = bufs[ci & 1][6]
        base = pl.multiple_of(w_base + ci * C, C)
        out_descs[ci] = pltpu.async_copy(o_v, out_hbm.at[pl.ds(base, C)],
                                         sem_out)
    for ci in sorted(out_descs):
        out_descs.pop(ci).wait()


def _tileview(a):
    return a.T.reshape(K // 8, 8, NWD, 128).transpose(0, 2, 1, 3)


def kernel(x, t, y, d):
    xf = x.reshape(N)
    mesh = plsc.VectorSubcoreMesh(
        core_axis_name="c", subcore_axis_name="s", num_cores=NC, num_subcores=NS
    )
    tbl = pltpu.VMEM((K // 8, CW, 8, 128), jnp.float32)
    vec = pltpu.VMEM((C,), jnp.float32)
    out = pl.kernel(
        _body,
        out_type=jax.ShapeDtypeStruct((N,), jnp.float32),
        mesh=mesh,
        compiler_params=pltpu.CompilerParams(needs_layout_passes=False),
        scratch_types=[
            tbl, tbl, tbl, vec, vec,
            tbl, tbl, tbl, vec, vec,
            pltpu.SemaphoreType.DMA, pltpu.SemaphoreType.DMA,
            pltpu.SemaphoreType.DMA, pltpu.SemaphoreType.DMA,
        ],
    )(xf, _tileview(t), _tileview(y), _tileview(d))
    return out[:, None]
